# baseline (speedup 1.0000x reference)
import functools

import jax
import jax.numpy as jnp
from jax import lax
from jax.experimental import pallas as pl
from jax.experimental.pallas import tpu as pltpu
from jax.experimental.pallas import tpu_sc as plsc

N_NODES = 10000
N_EDGES = 320000
D_NODE = 128
D_EDGE = 16
D_GLOBAL = 64
HIDDEN = 128

NUM_CORES = 2
NUM_SUBCORES = 16
NUM_WORKERS = NUM_CORES * NUM_SUBCORES

EDGES_PER_WORKER = N_EDGES // NUM_WORKERS
GATHER_CHUNK = 80
CHUNKS_PER_WORKER = EDGES_PER_WORKER // GATHER_CHUNK


def _precompute_body(nodes_ref, w1r_ref, w1s_ref, w1g_ref, glbls_ref, b1_ref,
                     p_ref, q_ref):
    c = jnp.dot(glbls_ref[...], w1g_ref[...],
                preferred_element_type=jnp.float32) + b1_ref[...]
    nodes = nodes_ref[...]
    p_ref[...] = jnp.dot(nodes, w1r_ref[...],
                         preferred_element_type=jnp.float32) + c
    q_ref[...] = jnp.dot(nodes, w1s_ref[...],
                         preferred_element_type=jnp.float32)


def _precompute(nodes, w1r, w1s, w1g, glbls, b1):
    return pl.pallas_call(
        _precompute_body,
        out_shape=(
            jax.ShapeDtypeStruct((N_NODES, HIDDEN), jnp.float32),
            jax.ShapeDtypeStruct((N_NODES, HIDDEN), jnp.float32),
        ),
    )(nodes, w1r, w1s, w1g, glbls, b1)


def _sc_gather_body(p_hbm, q_hbm, recv_hbm, send_hbm, gr_hbm, gs_hbm,
                    recv_v, send_v, rows_r, rows_s, sem_r, sem_s):
    wid = lax.axis_index("s") * NUM_CORES + lax.axis_index("c")
    base = wid * EDGES_PER_WORKER

    pltpu.sync_copy(recv_hbm.at[pl.ds(base, EDGES_PER_WORKER)], recv_v)
    pltpu.sync_copy(send_hbm.at[pl.ds(base, EDGES_PER_WORKER)], send_v)

    def chunk(j, carry):
        off = j * GATHER_CHUNK
        cr = pltpu.async_copy(
            p_hbm.at[recv_v.at[pl.ds(off, GATHER_CHUNK)]], rows_r, sem_r)
        cs = pltpu.async_copy(
            q_hbm.at[send_v.at[pl.ds(off, GATHER_CHUNK)]], rows_s, sem_s)
        cr.wait()
        pltpu.sync_copy(rows_r, gr_hbm.at[pl.ds(base + off, GATHER_CHUNK)])
        cs.wait()
        pltpu.sync_copy(rows_s, gs_hbm.at[pl.ds(base + off, GATHER_CHUNK)])
        return carry

    lax.fori_loop(0, CHUNKS_PER_WORKER, chunk, 0, unroll=False)


def _sc_gather(p, q, receivers, senders):
    mesh = plsc.VectorSubcoreMesh(core_axis_name="c", subcore_axis_name="s",
                                  num_cores=NUM_CORES,
                                  num_subcores=NUM_SUBCORES)
    return pl.kernel(
        _sc_gather_body,
        out_type=(
            jax.ShapeDtypeStruct((N_EDGES, HIDDEN), jnp.float32),
            jax.ShapeDtypeStruct((N_EDGES, HIDDEN), jnp.float32),
        ),
        mesh=mesh,
        scratch_types=[
            pltpu.VMEM((EDGES_PER_WORKER,), jnp.int32),
            pltpu.VMEM((EDGES_PER_WORKER,), jnp.int32),
            pltpu.VMEM((GATHER_CHUNK, HIDDEN), jnp.float32),
            pltpu.VMEM((GATHER_CHUNK, HIDDEN), jnp.float32),
            pltpu.SemaphoreType.DMA,
            pltpu.SemaphoreType.DMA,
        ],
        compiler_params=pltpu.CompilerParams(use_tc_tiling_on_sc=True),
    )(p, q, receivers, senders)


EDGE_BLOCK = 4000


def _mlp_body(gr_ref, gs_ref, edges_ref, w1e_ref, w2_ref, b2_ref, out_ref):
    h = gr_ref[...] + gs_ref[...] + jnp.dot(
        edges_ref[...], w1e_ref[...], preferred_element_type=jnp.float32)
    h = jnp.maximum(h, 0.0)
    out_ref[...] = jnp.dot(h, w2_ref[...],
                           preferred_element_type=jnp.float32) + b2_ref[...]


def _edge_mlp(gr, gs, edges, w1e, w2, b2):
    grid = (N_EDGES // EDGE_BLOCK,)
    return pl.pallas_call(
        _mlp_body,
        grid=grid,
        in_specs=[
            pl.BlockSpec((EDGE_BLOCK, HIDDEN), lambda i: (i, 0)),
            pl.BlockSpec((EDGE_BLOCK, HIDDEN), lambda i: (i, 0)),
            pl.BlockSpec((EDGE_BLOCK, D_EDGE), lambda i: (i, 0)),
            pl.BlockSpec((D_EDGE, HIDDEN), lambda i: (0, 0)),
            pl.BlockSpec((HIDDEN, D_EDGE), lambda i: (0, 0)),
            pl.BlockSpec((1, D_EDGE), lambda i: (0, 0)),
        ],
        out_specs=pl.BlockSpec((EDGE_BLOCK, D_EDGE), lambda i: (i, 0)),
        out_shape=jax.ShapeDtypeStruct((N_EDGES, D_EDGE), jnp.float32),
        compiler_params=pltpu.CompilerParams(
            dimension_semantics=("arbitrary",)),
    )(gr, gs, edges, w1e, w2, b2)


@jax.jit
def kernel(edges, nodes, glbls, W1, b1, W2, b2, senders, receivers):
    w1e = W1[:D_EDGE]
    w1r = W1[D_EDGE:D_EDGE + D_NODE]
    w1s = W1[D_EDGE + D_NODE:D_EDGE + 2 * D_NODE]
    w1g = W1[D_EDGE + 2 * D_NODE:]
    p, q = _precompute(nodes, w1r, w1s, w1g, glbls, b1.reshape(1, HIDDEN))
    gr, gs = _sc_gather(p, q, receivers, senders)
    return _edge_mlp(gr, gs, edges, w1e, W2, b2.reshape(1, D_EDGE))


# speedup vs baseline: 1.0534x; 1.0534x over previous
import functools

import jax
import jax.numpy as jnp
from jax import lax
from jax.experimental import pallas as pl
from jax.experimental.pallas import tpu as pltpu
from jax.experimental.pallas import tpu_sc as plsc

N_NODES = 10000
N_EDGES = 320000
D_NODE = 128
D_EDGE = 16
D_GLOBAL = 64
HIDDEN = 128

NUM_CORES = 2
NUM_SUBCORES = 16
NUM_WORKERS = NUM_CORES * NUM_SUBCORES

EDGES_PER_WORKER = N_EDGES // NUM_WORKERS
GATHER_CHUNK = 80
CHUNKS_PER_WORKER = EDGES_PER_WORKER // GATHER_CHUNK


def _precompute_body(nodes_ref, w1r_ref, w1s_ref, w1g_ref, glbls_ref, b1_ref,
                     p_ref, q_ref):
    c = jnp.dot(glbls_ref[...], w1g_ref[...],
                preferred_element_type=jnp.float32) + b1_ref[...]
    nodes = nodes_ref[...]
    p_ref[...] = jnp.dot(nodes, w1r_ref[...],
                         preferred_element_type=jnp.float32) + c
    q_ref[...] = jnp.dot(nodes, w1s_ref[...],
                         preferred_element_type=jnp.float32)


def _precompute(nodes, w1r, w1s, w1g, glbls, b1):
    return pl.pallas_call(
        _precompute_body,
        out_shape=(
            jax.ShapeDtypeStruct((N_NODES, HIDDEN), jnp.float32),
            jax.ShapeDtypeStruct((N_NODES, HIDDEN), jnp.float32),
        ),
    )(nodes, w1r, w1s, w1g, glbls, b1)


def _sc_gather_body(p_hbm, q_hbm, recv_hbm, send_hbm, gr_hbm, gs_hbm,
                    recv_v, send_v, rows_ra, rows_sa, rows_rb, rows_sb,
                    sem_ra, sem_sa, sem_rb, sem_sb):
    wid = lax.axis_index("s") * NUM_CORES + lax.axis_index("c")
    base = wid * EDGES_PER_WORKER

    pltpu.sync_copy(recv_hbm.at[pl.ds(base, EDGES_PER_WORKER)], recv_v)
    pltpu.sync_copy(send_hbm.at[pl.ds(base, EDGES_PER_WORKER)], send_v)

    def gather(j, rows_r, rows_s, sem_r, sem_s):
        off = j * GATHER_CHUNK
        cr = pltpu.async_copy(
            p_hbm.at[recv_v.at[pl.ds(off, GATHER_CHUNK)]], rows_r, sem_r)
        cs = pltpu.async_copy(
            q_hbm.at[send_v.at[pl.ds(off, GATHER_CHUNK)]], rows_s, sem_s)
        return cr, cs

    def drain_store(j, rows_r, rows_s, sem_r, sem_s):
        off = j * GATHER_CHUNK
        pltpu.make_async_copy(
            p_hbm.at[recv_v.at[pl.ds(off, GATHER_CHUNK)]], rows_r, sem_r
        ).wait()
        pltpu.sync_copy(rows_r, gr_hbm.at[pl.ds(base + off, GATHER_CHUNK)])
        pltpu.make_async_copy(
            q_hbm.at[send_v.at[pl.ds(off, GATHER_CHUNK)]], rows_s, sem_s
        ).wait()
        pltpu.sync_copy(rows_s, gs_hbm.at[pl.ds(base + off, GATHER_CHUNK)])

    gather(0, rows_ra, rows_sa, sem_ra, sem_sa)

    def body(i, carry):
        j = 2 * i
        gather(j + 1, rows_rb, rows_sb, sem_rb, sem_sb)
        drain_store(j, rows_ra, rows_sa, sem_ra, sem_sa)
        gather(j + 2, rows_ra, rows_sa, sem_ra, sem_sa)
        drain_store(j + 1, rows_rb, rows_sb, sem_rb, sem_sb)
        return carry

    lax.fori_loop(0, (CHUNKS_PER_WORKER - 1) // 2, body, 0, unroll=False)
    drain_store(CHUNKS_PER_WORKER - 1, rows_ra, rows_sa, sem_ra, sem_sa)


def _sc_gather(p, q, receivers, senders):
    mesh = plsc.VectorSubcoreMesh(core_axis_name="c", subcore_axis_name="s",
                                  num_cores=NUM_CORES,
                                  num_subcores=NUM_SUBCORES)
    return pl.kernel(
        _sc_gather_body,
        out_type=(
            jax.ShapeDtypeStruct((N_EDGES, HIDDEN), jnp.float32),
            jax.ShapeDtypeStruct((N_EDGES, HIDDEN), jnp.float32),
        ),
        mesh=mesh,
        scratch_types=[
            pltpu.VMEM((EDGES_PER_WORKER,), jnp.int32),
            pltpu.VMEM((EDGES_PER_WORKER,), jnp.int32),
            pltpu.VMEM((GATHER_CHUNK, HIDDEN), jnp.float32),
            pltpu.VMEM((GATHER_CHUNK, HIDDEN), jnp.float32),
            pltpu.VMEM((GATHER_CHUNK, HIDDEN), jnp.float32),
            pltpu.VMEM((GATHER_CHUNK, HIDDEN), jnp.float32),
            pltpu.SemaphoreType.DMA,
            pltpu.SemaphoreType.DMA,
            pltpu.SemaphoreType.DMA,
            pltpu.SemaphoreType.DMA,
        ],
        compiler_params=pltpu.CompilerParams(use_tc_tiling_on_sc=True),
    )(p, q, receivers, senders)


EDGE_BLOCK = 4000


def _mlp_body(gr_ref, gs_ref, edges_ref, w1e_ref, w2_ref, b2_ref, out_ref):
    h = gr_ref[...] + gs_ref[...] + jnp.dot(
        edges_ref[...], w1e_ref[...], preferred_element_type=jnp.float32)
    h = jnp.maximum(h, 0.0)
    out_ref[...] = jnp.dot(h, w2_ref[...],
                           preferred_element_type=jnp.float32) + b2_ref[...]


def _edge_mlp(gr, gs, edges, w1e, w2, b2):
    grid = (N_EDGES // EDGE_BLOCK,)
    return pl.pallas_call(
        _mlp_body,
        grid=grid,
        in_specs=[
            pl.BlockSpec((EDGE_BLOCK, HIDDEN), lambda i: (i, 0)),
            pl.BlockSpec((EDGE_BLOCK, HIDDEN), lambda i: (i, 0)),
            pl.BlockSpec((EDGE_BLOCK, D_EDGE), lambda i: (i, 0)),
            pl.BlockSpec((D_EDGE, HIDDEN), lambda i: (0, 0)),
            pl.BlockSpec((HIDDEN, D_EDGE), lambda i: (0, 0)),
            pl.BlockSpec((1, D_EDGE), lambda i: (0, 0)),
        ],
        out_specs=pl.BlockSpec((EDGE_BLOCK, D_EDGE), lambda i: (i, 0)),
        out_shape=jax.ShapeDtypeStruct((N_EDGES, D_EDGE), jnp.float32),
        compiler_params=pltpu.CompilerParams(
            dimension_semantics=("arbitrary",)),
    )(gr, gs, edges, w1e, w2, b2)


@jax.jit
def kernel(edges, nodes, glbls, W1, b1, W2, b2, senders, receivers):
    w1e = W1[:D_EDGE]
    w1r = W1[D_EDGE:D_EDGE + D_NODE]
    w1s = W1[D_EDGE + D_NODE:D_EDGE + 2 * D_NODE]
    w1g = W1[D_EDGE + 2 * D_NODE:]
    p, q = _precompute(nodes, w1r, w1s, w1g, glbls, b1.reshape(1, HIDDEN))
    gr, gs = _sc_gather(p, q, receivers, senders)
    return _edge_mlp(gr, gs, edges, w1e, W2, b2.reshape(1, D_EDGE))
